# baseline (speedup 1.0000x reference)
import jax
import jax.numpy as jnp
from jax.experimental import pallas as pl
from jax.experimental.pallas import tpu as pltpu


def _tile4_kernel(x_ref, o_ref):
    xb = x_ref[...]
    d = xb.shape[1]
    o_ref[:, 0 * d:1 * d] = xb
    o_ref[:, 1 * d:2 * d] = xb
    o_ref[:, 2 * d:3 * d] = xb
    o_ref[:, 3 * d:4 * d] = xb


def kernel(x, edge_index):
    del edge_index
    n, d = x.shape
    bn = 1000
    if n % bn != 0 or bn % 8 != 0:
        bn = n
    out = pl.pallas_call(
        _tile4_kernel,
        grid=(n // bn,),
        in_specs=[pl.BlockSpec((bn, d), lambda i: (i, 0))],
        out_specs=pl.BlockSpec((bn, 4 * d), lambda i: (i, 0)),
        out_shape=jax.ShapeDtypeStruct((n, 4 * d), x.dtype),
        compiler_params=pltpu.CompilerParams(
            dimension_semantics=("parallel",)),
    )(x)
    return out


# speedup vs baseline: 1.2215x; 1.2215x over previous
import jax
import jax.numpy as jnp
from jax.experimental import pallas as pl
from jax.experimental.pallas import tpu as pltpu


def _tile4_kernel(x_ref, o_ref):
    xb = x_ref[...]
    d = xb.shape[1]
    o_ref[:, 0 * d:1 * d] = xb
    o_ref[:, 1 * d:2 * d] = xb
    o_ref[:, 2 * d:3 * d] = xb
    o_ref[:, 3 * d:4 * d] = xb


def kernel(x, edge_index):
    del edge_index
    n, d = x.shape
    bn = 2000
    if n % bn != 0 or bn % 8 != 0:
        bn = n
    out = pl.pallas_call(
        _tile4_kernel,
        grid=(n // bn,),
        in_specs=[pl.BlockSpec((bn, d), lambda i: (i, 0))],
        out_specs=pl.BlockSpec((bn, 4 * d), lambda i: (i, 0)),
        out_shape=jax.ShapeDtypeStruct((n, 4 * d), x.dtype),
        compiler_params=pltpu.CompilerParams(
            dimension_semantics=("parallel",)),
    )(x)
    return out


# speedup vs baseline: 1.2384x; 1.0138x over previous
import jax
import jax.numpy as jnp
from jax.experimental import pallas as pl
from jax.experimental.pallas import tpu as pltpu


def _tile4_kernel(x_ref, o_ref):
    xb = x_ref[...]
    d = xb.shape[1]
    o_ref[:, 0 * d:1 * d] = xb
    o_ref[:, 1 * d:2 * d] = xb
    o_ref[:, 2 * d:3 * d] = xb
    o_ref[:, 3 * d:4 * d] = xb


def kernel(x, edge_index):
    del edge_index
    n, d = x.shape
    bn = 5000
    if n % bn != 0 or bn % 8 != 0:
        bn = n
    out = pl.pallas_call(
        _tile4_kernel,
        grid=(n // bn,),
        in_specs=[pl.BlockSpec((bn, d), lambda i: (i, 0))],
        out_specs=pl.BlockSpec((bn, 4 * d), lambda i: (i, 0)),
        out_shape=jax.ShapeDtypeStruct((n, 4 * d), x.dtype),
        compiler_params=pltpu.CompilerParams(
            dimension_semantics=("parallel",)),
    )(x)
    return out
